# baseline (speedup 1.0000x reference)
import functools

import jax
import jax.numpy as jnp
from jax import lax
from jax.experimental import pallas as pl
from jax.experimental.pallas import tpu as pltpu
from jax.experimental.pallas import tpu_sc as plsc

HIDDEN = 1024
B = 4 * 4096
K = 4
NQ = B // K
QW = K * HIDDEN // 2
NC = 2
NS = 16
NW = NC * NS
QPW = NQ // NW
CH = 16
NCHUNK = QPW // CH

_mesh = plsc.VectorSubcoreMesh(core_axis_name="c", subcore_axis_name="s")


@functools.partial(
    pl.kernel,
    out_type=jax.ShapeDtypeStruct((NQ, QW), jnp.int32),
    mesh=_mesh,
    scratch_types=[
        pltpu.VMEM((QPW,), jnp.int32),
        pltpu.VMEM((CH, QW), jnp.int32),
        pltpu.VMEM((CH, QW), jnp.int32),
        pltpu.SemaphoreType.DMA,
        pltpu.SemaphoreType.DMA,
        pltpu.SemaphoreType.DMA,
        pltpu.SemaphoreType.DMA,
    ],
)
def _lookup(combo_hbm, qtab_hbm, out_hbm, idx_v, buf0, buf1, g0, g1, s0, s1):
    wid = lax.axis_index("s") * NC + lax.axis_index("c")
    qbase = wid * QPW
    pltpu.sync_copy(combo_hbm.at[pl.ds(qbase, QPW)], idx_v)

    bufs = (buf0, buf1)
    gsems = (g0, g1)
    ssems = (s0, s1)

    def start_gather(c):
        b = c % 2
        return pltpu.async_copy(
            qtab_hbm.at[idx_v.at[pl.ds(c * CH, CH)]], bufs[b], gsems[b])

    def start_out(c):
        b = c % 2
        return pltpu.async_copy(
            bufs[b], out_hbm.at[pl.ds(qbase + c * CH, CH)], ssems[b])

    gathers = [None] * NCHUNK
    outs = [None] * NCHUNK
    gathers[0] = start_gather(0)
    for c in range(NCHUNK):
        gathers[c].wait()
        outs[c] = start_out(c)
        if c + 1 < NCHUNK:
            if c >= 1:
                outs[c - 1].wait()
            gathers[c + 1] = start_gather(c + 1)
    outs[NCHUNK - 2].wait()
    outs[NCHUNK - 1].wait()


def kernel(token_type_ids, token_type_table):
    ids = jnp.reshape(token_type_ids, (B,)).astype(jnp.int32)
    quads = jnp.reshape(ids, (NQ, K))
    combo = (quads[:, 0] + 2 * quads[:, 1] + 4 * quads[:, 2]
             + 8 * quads[:, 3])
    c = jnp.arange(16, dtype=jnp.int32)[:, None]
    sel = jnp.reshape((c >> jnp.arange(K, dtype=jnp.int32)[None, :]) & 1,
                      (-1,))
    qtab = jnp.reshape(token_type_table[sel, :], (16, K * HIDDEN))
    qtab_w = lax.bitcast_convert_type(
        jnp.reshape(qtab, (16, QW, 2)), jnp.int32)
    out_w = _lookup(combo, qtab_w)
    out = lax.bitcast_convert_type(out_w, jnp.float16)
    return jnp.reshape(out, (B, HIDDEN))


# speedup vs baseline: 1.0119x; 1.0119x over previous
import functools

import jax
import jax.numpy as jnp
from jax import lax
from jax.experimental import pallas as pl
from jax.experimental.pallas import tpu as pltpu
from jax.experimental.pallas import tpu_sc as plsc

HIDDEN = 1024
B = 4 * 4096
K = 4
NQ = B // K
QW = K * HIDDEN // 2
NCOMBO = 1 << K
NC = 2
NS = 16
NW = NC * NS
QPW = NQ // NW
VL = 16

_mesh = plsc.VectorSubcoreMesh(core_axis_name="c", subcore_axis_name="s")


@functools.partial(
    pl.kernel,
    out_type=jax.ShapeDtypeStruct((NQ, QW), jnp.int32),
    mesh=_mesh,
    scratch_types=[
        pltpu.VMEM((QPW,), jnp.int32),
        pltpu.VMEM((NCOMBO, QW), jnp.int32),
        pltpu.SemaphoreType.DMA,
    ],
)
def _lookup(combo_hbm, qtab_hbm, out_hbm, idx_v, qtab_v, sem):
    wid = lax.axis_index("s") * NC + lax.axis_index("c")
    qbase = wid * QPW
    pltpu.sync_copy(combo_hbm.at[pl.ds(qbase, QPW)], idx_v)
    pltpu.sync_copy(qtab_hbm, qtab_v)

    copies = []
    for g in range(QPW // VL):
        cv = idx_v[pl.ds(g * VL, VL)]
        for j in range(VL):
            q = g * VL + j
            copies.append(pltpu.async_copy(
                qtab_v.at[pl.ds(cv[j], 1)],
                out_hbm.at[pl.ds(qbase + q, 1)],
                sem))
    for cp in copies:
        cp.wait()


def kernel(token_type_ids, token_type_table):
    ids = jnp.reshape(token_type_ids, (B,)).astype(jnp.int32)
    quads = jnp.reshape(ids, (NQ, K))
    combo = (quads[:, 0] + 2 * quads[:, 1] + 4 * quads[:, 2]
             + 8 * quads[:, 3])
    c = jnp.arange(NCOMBO, dtype=jnp.int32)[:, None]
    sel = jnp.reshape((c >> jnp.arange(K, dtype=jnp.int32)[None, :]) & 1,
                      (-1,))
    qtab = jnp.reshape(token_type_table[sel, :], (NCOMBO, K * HIDDEN))
    qtab_w = lax.bitcast_convert_type(
        jnp.reshape(qtab, (NCOMBO, QW, 2)), jnp.int32)
    out_w = _lookup(combo, qtab_w)
    out = lax.bitcast_convert_type(out_w, jnp.float16)
    return jnp.reshape(out, (B, HIDDEN))


# speedup vs baseline: 28.5404x; 28.2048x over previous
import functools

import jax
import jax.numpy as jnp
from jax import lax
from jax.experimental import pallas as pl
from jax.experimental.pallas import tpu as pltpu
from jax.experimental.pallas import tpu_sc as plsc

HIDDEN = 1024
DW = HIDDEN // 2
B = 4 * 4096
K = 4
NQ = B // K
NCOMBO = 1 << K
NC = 2
NS = 16
NW = NC * NS
QPW = NQ // NW
VL = 16

_mesh = plsc.VectorSubcoreMesh(core_axis_name="c", subcore_axis_name="s")


@functools.partial(
    pl.kernel,
    out_type=jax.ShapeDtypeStruct((B, DW), jnp.int32),
    mesh=_mesh,
    scratch_types=[
        pltpu.VMEM((QPW,), jnp.int32),
        pltpu.VMEM((NCOMBO * K, DW), jnp.int32),
        pltpu.SemaphoreType.DMA,
    ],
)
def _lookup(combo_hbm, qtab_hbm, out_hbm, idx_v, qtab_v, sem):
    wid = lax.axis_index("s") * NC + lax.axis_index("c")
    qbase = wid * QPW
    rbase = qbase * K
    pltpu.sync_copy(combo_hbm.at[pl.ds(qbase, QPW)], idx_v)
    pltpu.sync_copy(qtab_hbm, qtab_v)

    copies = []
    for g in range(QPW // VL):
        cv = idx_v[pl.ds(g * VL, VL)]
        for j in range(VL):
            q = g * VL + j
            copies.append(pltpu.async_copy(
                qtab_v.at[pl.ds(cv[j] * K, K)],
                out_hbm.at[pl.ds(rbase + q * K, K)],
                sem))
    for cp in copies:
        cp.wait()


def kernel(token_type_ids, token_type_table):
    ids = jnp.reshape(token_type_ids, (B,)).astype(jnp.int32)
    quads = jnp.reshape(ids, (NQ, K))
    combo = (quads[:, 0] + 2 * quads[:, 1] + 4 * quads[:, 2]
             + 8 * quads[:, 3])
    c = jnp.arange(NCOMBO, dtype=jnp.int32)[:, None]
    sel = jnp.reshape((c >> jnp.arange(K, dtype=jnp.int32)[None, :]) & 1,
                      (-1,))
    qtab = token_type_table[sel, :]
    qtab_w = lax.bitcast_convert_type(
        jnp.reshape(qtab, (NCOMBO * K, DW, 2)), jnp.int32)
    out_w = _lookup(combo, qtab_w)
    out = lax.bitcast_convert_type(out_w, jnp.float16)
    return jnp.reshape(out, (B, HIDDEN))
